# baseline (speedup 1.0000x reference)
import functools

import jax
import jax.numpy as jnp
from jax import lax
from jax.experimental import pallas as pl
from jax.experimental.pallas import tpu as pltpu
from jax.experimental.pallas import tpu_sc as plsc

B = 4
S = 2048
D = 768
L = 16
NC = 2
NS = 16
NW = NC * NS
SPW = S // NW
HC = 32
DV = D // L

_mesh = plsc.VectorSubcoreMesh(core_axis_name="c", subcore_axis_name="s")


@functools.partial(
    pl.kernel,
    mesh=_mesh,
    out_type=jax.ShapeDtypeStruct((B * S, D), jnp.float32),
    scratch_types=[
        pltpu.VMEM((HC,), jnp.int32),
        pltpu.VMEM((HC,), jnp.int32),
        pltpu.VMEM((HC, D), jnp.float32),
        pltpu.VMEM((HC, D), jnp.float32),
        pltpu.VMEM((SPW, D), jnp.float32),
        pltpu.SemaphoreType.DMA,
    ],
)
def _emb_kernel(ids_hbm, tt_hbm, word_hbm, pos_hbm, type_hbm, out_hbm,
                ids_v, tt_v, w_v, tr_v, p_v, sem):
    wid = lax.axis_index("s") * NC + lax.axis_index("c")
    s0 = wid * SPW
    pltpu.sync_copy(pos_hbm.at[pl.ds(s0, SPW), :], p_v)
    for b in range(B):
        for h in range(SPW // HC):
            base = b * S + s0 + h * HC
            pltpu.sync_copy(ids_hbm.at[pl.ds(base, HC)], ids_v)
            pltpu.sync_copy(tt_hbm.at[pl.ds(base, HC)], tt_v)
            cw = pltpu.async_copy(word_hbm.at[ids_v], w_v, sem)
            ct = pltpu.async_copy(type_hbm.at[tt_v], tr_v, sem)
            cw.wait()
            ct.wait()

            def body(i, _, h=h):
                for d in range(DV):
                    dsl = pl.ds(d * L, L)
                    w_v[i, dsl] = w_v[i, dsl] + tr_v[i, dsl] + p_v[h * HC + i, dsl]
                return _

            lax.fori_loop(0, HC, body, None)
            pltpu.sync_copy(w_v, out_hbm.at[pl.ds(base, HC), :])


def kernel(input_ids, token_type_ids, word_emb, pos_emb, type_emb):
    ids = input_ids.reshape(-1).astype(jnp.int32)
    tt = token_type_ids.reshape(-1).astype(jnp.int32)
    out = _emb_kernel(ids, tt, word_emb, pos_emb, type_emb)
    return out.reshape(B, S, D)
